# baseline (speedup 1.0000x reference)
import functools

import jax
import jax.numpy as jnp
import numpy as np
from jax import lax
from jax.experimental import pallas as pl
from jax.experimental.pallas import tpu as pltpu
from jax.experimental.pallas import tpu_sc as plsc

N = 10000
K = 32
RADIAL = 256
HIDDEN = 128
OUT = 64
E = N * K
LOG2 = float(np.log(2.0))

_CH = 80
_GCH = 25


def _sc_gather(table, idx):
    info = plsc.get_sparse_core_info()
    nw = info.num_cores * info.num_subcores
    e_per_w = E // nw
    ngroup = e_per_w // (_CH * _GCH)
    mesh = plsc.VectorSubcoreMesh(core_axis_name="c", subcore_axis_name="s")

    @functools.partial(
        pl.kernel,
        mesh=mesh,
        out_type=jax.ShapeDtypeStruct((E, 2 * OUT), jnp.float32),
        scratch_types=[
            pltpu.VMEM((e_per_w,), jnp.int32),
            pltpu.VMEM((_CH, 2 * OUT), jnp.float32),
            pltpu.VMEM((_CH, 2 * OUT), jnp.float32),
            pltpu.SemaphoreType.DMA,
            pltpu.SemaphoreType.DMA,
            pltpu.SemaphoreType.DMA,
            pltpu.SemaphoreType.DMA,
        ],
    )
    def gather_kernel(table_hbm, idx_hbm, out_hbm, idx_v, rows0, rows1,
                      gsem0, gsem1, ssem0, ssem1):
        wid = lax.axis_index("s") * info.num_cores + lax.axis_index("c")
        base = wid * e_per_w
        pltpu.sync_copy(idx_hbm.at[pl.ds(base, e_per_w)], idx_v)

        rows = (rows0, rows1)
        gsem = (gsem0, gsem1)
        ssem = (ssem0, ssem1)

        def group(gi, carry):
            goff = gi * (_CH * _GCH)
            gath = [None, None]
            scat = [None, None]
            for g in range(_GCH):
                b = g & 1
                off = goff + g * _CH
                if g >= 2:
                    scat[b].wait()
                gath[b] = pltpu.async_copy(
                    table_hbm.at[idx_v.at[pl.ds(off, _CH)]], rows[b], gsem[b])
                if g >= 1:
                    p = 1 - b
                    gath[p].wait()
                    scat[p] = pltpu.async_copy(
                        rows[p],
                        out_hbm.at[pl.ds(base + off - _CH, _CH)], ssem[p])
            last = (_GCH - 1) & 1
            gath[last].wait()
            scat[last] = pltpu.async_copy(
                rows[last],
                out_hbm.at[pl.ds(base + goff + (_GCH - 1) * _CH, _CH)],
                ssem[last])
            scat[0].wait()
            scat[1].wait()
            return carry

        lax.fori_loop(0, ngroup, group, 0)

    return gather_kernel(table, idx)


_BN = 200
_EB = _BN * K
_GRID = N // _BN


def _ssp(x):
    return jnp.maximum(x, 0.0) + jnp.log1p(jnp.exp(-jnp.abs(x))) - LOG2


def _tc_body(bf_ref, hs_ref, he_ref, co_ref,
             w1_ref, b1_ref, w2_ref, b2_ref, w3_ref, b3_ref, w4_ref, b4_ref,
             out_ref):
    x = bf_ref[...]
    x = _ssp(jnp.dot(x, w1_ref[...],
                     preferred_element_type=jnp.float32) + b1_ref[...])
    x = _ssp(jnp.dot(x, w2_ref[...],
                     preferred_element_type=jnp.float32) + b2_ref[...])
    hs = hs_ref[...][:, :OUT]
    v = x * hs * he_ref[...] * co_ref[...]
    m = jnp.sum(v.reshape(_BN, K, OUT), axis=1)
    m = _ssp(jnp.dot(m, w3_ref[...],
                     preferred_element_type=jnp.float32) + b3_ref[...])
    out_ref[...] = _ssp(jnp.dot(m, w4_ref[...],
                                preferred_element_type=jnp.float32) + b4_ref[...])


def _tc_conv(bf2, h_src, he2, co2, W1, b1, W2, b2, W3, b3, W4, b4):
    edge_spec = lambda w: pl.BlockSpec((_EB, w), lambda i: (i, 0))
    full_spec = lambda a: pl.BlockSpec(a.shape, lambda i: (0,) * a.ndim)
    return pl.pallas_call(
        _tc_body,
        grid=(_GRID,),
        in_specs=[
            edge_spec(RADIAL),
            edge_spec(2 * OUT),
            edge_spec(OUT),
            edge_spec(1),
            full_spec(W1), full_spec(b1),
            full_spec(W2), full_spec(b2),
            full_spec(W3), full_spec(b3),
            full_spec(W4), full_spec(b4),
        ],
        out_specs=pl.BlockSpec((_BN, OUT), lambda i: (i, 0)),
        out_shape=jax.ShapeDtypeStruct((N, OUT), jnp.float32),
        compiler_params=pltpu.CompilerParams(
            dimension_semantics=("arbitrary",),
        ),
    )(bf2, h_src, he2, co2, W1, b1, W2, b2, W3, b3, W4, b4)


def kernel(bf, h, knn, h_edge, cutoff, W1, b1, W2, b2, W3, b3, W4, b4):
    idx = knn.reshape(-1).astype(jnp.int32)
    h128 = jnp.pad(h, ((0, 0), (0, OUT)))
    h_src = _sc_gather(h128, idx)
    bf2 = bf.reshape(E, RADIAL)
    he2 = h_edge.reshape(E, OUT)
    co2 = cutoff.reshape(E, 1)
    return _tc_conv(bf2, h_src, he2, co2,
                    W1, b1.reshape(1, HIDDEN), W2, b2.reshape(1, OUT),
                    W3, b3.reshape(1, OUT), W4, b4.reshape(1, OUT))


# speedup vs baseline: 1.0993x; 1.0993x over previous
import functools

import jax
import jax.numpy as jnp
import numpy as np
from jax import lax
from jax.experimental import pallas as pl
from jax.experimental.pallas import tpu as pltpu
from jax.experimental.pallas import tpu_sc as plsc

N = 10000
K = 32
RADIAL = 256
HIDDEN = 128
OUT = 64
E = N * K
LOG2 = float(np.log(2.0))

_CH = 80
_GCH = 25


def _sc_gather(table, idx):
    info = plsc.get_sparse_core_info()
    nw = info.num_cores * info.num_subcores
    e_per_w = E // nw
    ngroup = e_per_w // (_CH * _GCH)
    mesh = plsc.VectorSubcoreMesh(core_axis_name="c", subcore_axis_name="s")

    @functools.partial(
        pl.kernel,
        mesh=mesh,
        out_type=jax.ShapeDtypeStruct((E, 2 * OUT), jnp.float32),
        scratch_types=[
            pltpu.VMEM((e_per_w,), jnp.int32),
            pltpu.VMEM((_CH, OUT), jnp.float32),
            pltpu.VMEM((_CH, OUT), jnp.float32),
            pltpu.SemaphoreType.DMA,
            pltpu.SemaphoreType.DMA,
            pltpu.SemaphoreType.DMA,
            pltpu.SemaphoreType.DMA,
        ],
        compiler_params=pltpu.CompilerParams(use_tc_tiling_on_sc=False),
    )
    def gather_kernel(table_hbm, idx_hbm, out_hbm, idx_v, rows0, rows1,
                      gsem0, gsem1, ssem0, ssem1):
        wid = lax.axis_index("s") * info.num_cores + lax.axis_index("c")
        base = wid * e_per_w
        pltpu.sync_copy(idx_hbm.at[pl.ds(base, e_per_w)], idx_v)

        rows = (rows0, rows1)
        gsem = (gsem0, gsem1)
        ssem = (ssem0, ssem1)

        def group(gi, carry):
            goff = gi * (_CH * _GCH)
            gath = [None, None]
            scat = [None, None]
            for g in range(_GCH):
                b = g & 1
                off = goff + g * _CH
                if g >= 2:
                    scat[b].wait()
                gath[b] = pltpu.async_copy(
                    table_hbm.at[idx_v.at[pl.ds(off, _CH)]], rows[b], gsem[b])
                if g >= 1:
                    p = 1 - b
                    gath[p].wait()
                    scat[p] = pltpu.async_copy(
                        rows[p],
                        out_hbm.at[pl.ds(base + off - _CH, _CH),
                                   pl.ds(0, OUT)], ssem[p])
            last = (_GCH - 1) & 1
            gath[last].wait()
            scat[last] = pltpu.async_copy(
                rows[last],
                out_hbm.at[pl.ds(base + goff + (_GCH - 1) * _CH, _CH),
                           pl.ds(0, OUT)],
                ssem[last])
            scat[0].wait()
            scat[1].wait()
            return carry

        lax.fori_loop(0, ngroup, group, 0)

    return gather_kernel(table, idx)


_BN = 200
_EB = _BN * K
_GRID = N // _BN


def _ssp(x):
    return jnp.maximum(x, 0.0) + jnp.log1p(jnp.exp(-jnp.abs(x))) - LOG2


def _tc_body(bf_ref, hs_ref, he_ref, co_ref,
             w1_ref, b1_ref, w2_ref, b2_ref, w3_ref, b3_ref, w4_ref, b4_ref,
             out_ref):
    x = bf_ref[...]
    x = _ssp(jnp.dot(x, w1_ref[...],
                     preferred_element_type=jnp.float32) + b1_ref[...])
    x = _ssp(jnp.dot(x, w2_ref[...],
                     preferred_element_type=jnp.float32) + b2_ref[...])
    hs = hs_ref[...].reshape(_EB, 2 * OUT)[:, :OUT]
    v = x * hs * he_ref[...] * co_ref[...]
    m = jnp.sum(v.reshape(_BN, K, OUT), axis=1)
    m = _ssp(jnp.dot(m, w3_ref[...],
                     preferred_element_type=jnp.float32) + b3_ref[...])
    out_ref[...] = _ssp(jnp.dot(m, w4_ref[...],
                                preferred_element_type=jnp.float32) + b4_ref[...])


def _tc_conv(bf2, hs_flat, he2, cutoff, W1, b1, W2, b2, W3, b3, W4, b4):
    edge_spec = lambda w: pl.BlockSpec((_EB, w), lambda i: (i, 0))
    full_spec = lambda a: pl.BlockSpec(a.shape, lambda i: (0,) * a.ndim)
    return pl.pallas_call(
        _tc_body,
        grid=(_GRID,),
        in_specs=[
            edge_spec(RADIAL),
            pl.BlockSpec((_EB * 2 * OUT,), lambda i: (i,)),
            edge_spec(OUT),
            edge_spec(1),
            full_spec(W1), full_spec(b1),
            full_spec(W2), full_spec(b2),
            full_spec(W3), full_spec(b3),
            full_spec(W4), full_spec(b4),
        ],
        out_specs=pl.BlockSpec((_BN, OUT), lambda i: (i, 0)),
        out_shape=jax.ShapeDtypeStruct((N, OUT), jnp.float32),
        compiler_params=pltpu.CompilerParams(
            dimension_semantics=("arbitrary",),
        ),
    )(bf2, hs_flat, he2, cutoff, W1, b1, W2, b2, W3, b3, W4, b4)


def kernel(bf, h, knn, h_edge, cutoff, W1, b1, W2, b2, W3, b3, W4, b4):
    idx = knn.reshape(-1).astype(jnp.int32)
    h_src = _sc_gather(h, idx)
    bf2 = bf.reshape(E, RADIAL)
    he2 = h_edge.reshape(E, OUT)
    co2 = cutoff.reshape(E, 1)
    return _tc_conv(bf2, h_src.reshape(-1), he2, co2,
                    W1, b1.reshape(1, HIDDEN), W2, b2.reshape(1, OUT),
                    W3, b3.reshape(1, OUT), W4, b4.reshape(1, OUT))


# speedup vs baseline: 1.2552x; 1.1419x over previous
import functools

import jax
import jax.numpy as jnp
import numpy as np
from jax import lax
from jax.experimental import pallas as pl
from jax.experimental.pallas import tpu as pltpu
from jax.experimental.pallas import tpu_sc as plsc

N = 10000
K = 32
RADIAL = 256
HIDDEN = 128
OUT = 64
E = N * K
LOG2 = float(np.log(2.0))

_CH = 80
_GCH = 25


def _sc_gather(table, idx):
    info = plsc.get_sparse_core_info()
    nw = info.num_cores * info.num_subcores
    e_per_w = E // nw
    ngroup = e_per_w // (_CH * _GCH)
    mesh = plsc.VectorSubcoreMesh(core_axis_name="c", subcore_axis_name="s")

    @functools.partial(
        pl.kernel,
        mesh=mesh,
        out_type=jax.ShapeDtypeStruct((E, 2 * OUT), jnp.float32),
        scratch_types=[
            pltpu.VMEM((e_per_w,), jnp.int32),
            pltpu.VMEM((_CH, OUT), jnp.float32),
            pltpu.VMEM((_CH, OUT), jnp.float32),
            pltpu.SemaphoreType.DMA,
            pltpu.SemaphoreType.DMA,
            pltpu.SemaphoreType.DMA,
            pltpu.SemaphoreType.DMA,
        ],
        compiler_params=pltpu.CompilerParams(use_tc_tiling_on_sc=False),
    )
    def gather_kernel(table_hbm, idx_hbm, out_hbm, idx_v, rows0, rows1,
                      gsem0, gsem1, ssem0, ssem1):
        wid = lax.axis_index("s") * info.num_cores + lax.axis_index("c")
        base = wid * e_per_w
        pltpu.sync_copy(idx_hbm.at[pl.ds(base, e_per_w)], idx_v)

        rows = (rows0, rows1)
        gsem = (gsem0, gsem1)
        ssem = (ssem0, ssem1)

        def group(gi, carry):
            goff = gi * (_CH * _GCH)
            gath = [None, None]
            scat = [None, None]
            for g in range(_GCH):
                b = g & 1
                off = goff + g * _CH
                if g >= 2:
                    scat[b].wait()
                gath[b] = pltpu.async_copy(
                    table_hbm.at[idx_v.at[pl.ds(off, _CH)]], rows[b], gsem[b])
                if g >= 1:
                    p = 1 - b
                    gath[p].wait()
                    scat[p] = pltpu.async_copy(
                        rows[p],
                        out_hbm.at[pl.ds(base + off - _CH, _CH),
                                   pl.ds(0, OUT)], ssem[p])
            last = (_GCH - 1) & 1
            gath[last].wait()
            scat[last] = pltpu.async_copy(
                rows[last],
                out_hbm.at[pl.ds(base + goff + (_GCH - 1) * _CH, _CH),
                           pl.ds(0, OUT)],
                ssem[last])
            scat[0].wait()
            scat[1].wait()
            return carry

        lax.fori_loop(0, ngroup, group, 0)

    return gather_kernel(table, idx)


_BN = 200
_EB = _BN * K
_GRID = N // _BN


def _ssp(x):
    return jnp.maximum(x, 0.0) + jnp.log1p(jnp.exp(-jnp.abs(x))) - LOG2


def _tc_body(bf_ref, hs_ref, he_ref, co_ref,
             w1_ref, b1_ref, w2_ref, b2_ref, w3_ref, b3_ref, w4_ref, b4_ref,
             out_ref):
    x = bf_ref[...]
    x = _ssp(jnp.dot(x, w1_ref[...],
                     preferred_element_type=jnp.float32) + b1_ref[...])
    x = _ssp(jnp.dot(x, w2_ref[...],
                     preferred_element_type=jnp.float32) + b2_ref[...])
    hs = hs_ref[...].reshape(_EB, 2 * OUT)[:, :OUT]
    v = (x * hs * he_ref[...]).reshape(_BN, K, OUT)
    v = v * co_ref[...][:, :, None]
    m = jnp.sum(v, axis=1)
    m = _ssp(jnp.dot(m, w3_ref[...],
                     preferred_element_type=jnp.float32) + b3_ref[...])
    out_ref[...] = _ssp(jnp.dot(m, w4_ref[...],
                                preferred_element_type=jnp.float32) + b4_ref[...])


def _tc_conv(bf2, hs_flat, he2, cutoff, W1, b1, W2, b2, W3, b3, W4, b4):
    edge_spec = lambda w: pl.BlockSpec((_EB, w), lambda i: (i, 0))
    full_spec = lambda a: pl.BlockSpec(a.shape, lambda i: (0,) * a.ndim)
    return pl.pallas_call(
        _tc_body,
        grid=(_GRID,),
        in_specs=[
            edge_spec(RADIAL),
            pl.BlockSpec((_EB * 2 * OUT,), lambda i: (i,)),
            edge_spec(OUT),
            pl.BlockSpec((_BN, K), lambda i: (i, 0)),
            full_spec(W1), full_spec(b1),
            full_spec(W2), full_spec(b2),
            full_spec(W3), full_spec(b3),
            full_spec(W4), full_spec(b4),
        ],
        out_specs=pl.BlockSpec((_BN, OUT), lambda i: (i, 0)),
        out_shape=jax.ShapeDtypeStruct((N, OUT), jnp.float32),
        compiler_params=pltpu.CompilerParams(
            dimension_semantics=("arbitrary",),
        ),
    )(bf2, hs_flat, he2, cutoff, W1, b1, W2, b2, W3, b3, W4, b4)


def kernel(bf, h, knn, h_edge, cutoff, W1, b1, W2, b2, W3, b3, W4, b4):
    idx = knn.reshape(-1).astype(jnp.int32)
    h_src = _sc_gather(h, idx)
    bf2 = bf.reshape(E, RADIAL)
    he2 = h_edge.reshape(E, OUT)
    return _tc_conv(bf2, h_src.reshape(-1), he2, cutoff,
                    W1, b1.reshape(1, HIDDEN), W2, b2.reshape(1, OUT),
                    W3, b3.reshape(1, OUT), W4, b4.reshape(1, OUT))
